# baseline (speedup 1.0000x reference)
import numpy as np

import jax
import jax.numpy as jnp
from jax.experimental import pallas as pl
from jax.experimental.pallas import tpu as pltpu

N = 10000
NFEAT = 128
NHID = 64
NCLASS = 16
BLK = 1000
NB = N // BLK


def _pass1_kernel(x_ref, adj_ref, w1_ref, b1_ref, w2_ref,
                  part_ref, s2out_ref, s1_ref, s2s_ref, hacc_ref):
    i = pl.program_id(0)
    c = pl.program_id(1)

    @pl.when(jnp.logical_and(i == 0, c == 0))
    def _():
        s1_ref[...] = jnp.dot(x_ref[...], w1_ref[...],
                              preferred_element_type=jnp.float32)
        s2s_ref[...] = jnp.zeros_like(s2s_ref)

    blk = adj_ref[:, 0, 0, :]
    h_c = jnp.dot(blk, s1_ref[pl.ds(c * BLK, BLK), :],
                  preferred_element_type=jnp.float32)
    p_c = jnp.dot(blk, s2s_ref[pl.ds(c * BLK, BLK), :],
                  preferred_element_type=jnp.float32)

    @pl.when(c == 0)
    def _():
        hacc_ref[...] = h_c
        part_ref[...] = p_c

    @pl.when(c != 0)
    def _():
        hacc_ref[...] += h_c
        part_ref[...] += p_c

    @pl.when(c == NB - 1)
    def _():
        h = jnp.maximum(hacc_ref[...] + b1_ref[...], 0.0)
        s2_i = jnp.dot(h, w2_ref[...], preferred_element_type=jnp.float32)
        s2s_ref[pl.ds(i * BLK, BLK), :] = s2_i
        s2out_ref[...] = s2_i


def _pass2_kernel(i_ref, c_ref, adj_ref, s2_ref, part_ref, b2_ref, o_ref):
    t = pl.program_id(0)
    i = i_ref[t]
    c = c_ref[t]
    contrib = jnp.dot(adj_ref[:, 0, 0, :], s2_ref[...],
                      preferred_element_type=jnp.float32)

    @pl.when(c == i)
    def _():
        o_ref[...] = part_ref[...] + contrib

    @pl.when(c != i)
    def _():
        o_ref[...] += contrib

    @pl.when(c == NB - 1)
    def _():
        z = o_ref[...] + b2_ref[...]
        m = jnp.max(z, axis=1, keepdims=True)
        shifted = z - m
        lse = jnp.log(jnp.sum(jnp.exp(shifted), axis=1, keepdims=True))
        o_ref[...] = shifted - lse


_PAIRS = [(i, c) for i in range(NB) for c in range(i, NB)]
_I_ARR = np.array([p[0] for p in _PAIRS], dtype=np.int32)
_C_ARR = np.array([p[1] for p in _PAIRS], dtype=np.int32)
_T = len(_PAIRS)


@jax.jit
def kernel(x, adj, W1, b1, W2, b2):
    b1r = b1.reshape(1, NHID)
    b2r = b2.reshape(1, NCLASS)
    adj4 = adj.reshape(N, NB, 1, BLK)

    part, s2 = pl.pallas_call(
        _pass1_kernel,
        grid=(NB, NB),
        in_specs=[
            pl.BlockSpec((N, NFEAT), lambda i, c: (0, 0)),
            pl.BlockSpec((BLK, 1, 1, BLK), lambda i, c: (i, c, 0, 0)),
            pl.BlockSpec((NFEAT, NHID), lambda i, c: (0, 0)),
            pl.BlockSpec((1, NHID), lambda i, c: (0, 0)),
            pl.BlockSpec((NHID, NCLASS), lambda i, c: (0, 0)),
        ],
        out_specs=[
            pl.BlockSpec((BLK, NCLASS), lambda i, c: (i, 0)),
            pl.BlockSpec((BLK, NCLASS), lambda i, c: (i, 0)),
        ],
        out_shape=[
            jax.ShapeDtypeStruct((N, NCLASS), jnp.float32),
            jax.ShapeDtypeStruct((N, NCLASS), jnp.float32),
        ],
        scratch_shapes=[
            pltpu.VMEM((N, NHID), jnp.float32),
            pltpu.VMEM((N, NCLASS), jnp.float32),
            pltpu.VMEM((BLK, NHID), jnp.float32),
        ],
        compiler_params=pltpu.CompilerParams(
            dimension_semantics=("arbitrary", "arbitrary"),
        ),
    )(x, adj4, W1, b1r, W2)

    out = pl.pallas_call(
        _pass2_kernel,
        grid_spec=pltpu.PrefetchScalarGridSpec(
            num_scalar_prefetch=2,
            grid=(_T,),
            in_specs=[
                pl.BlockSpec((BLK, 1, 1, BLK),
                             lambda t, i_ref, c_ref: (i_ref[t], c_ref[t],
                                                      0, 0)),
                pl.BlockSpec((BLK, NCLASS),
                             lambda t, i_ref, c_ref: (c_ref[t], 0)),
                pl.BlockSpec((BLK, NCLASS),
                             lambda t, i_ref, c_ref: (i_ref[t], 0)),
                pl.BlockSpec((1, NCLASS),
                             lambda t, i_ref, c_ref: (0, 0)),
            ],
            out_specs=pl.BlockSpec(
                (BLK, NCLASS), lambda t, i_ref, c_ref: (i_ref[t], 0)),
        ),
        out_shape=jax.ShapeDtypeStruct((N, NCLASS), jnp.float32),
        compiler_params=pltpu.CompilerParams(
            dimension_semantics=("arbitrary",),
        ),
    )(jnp.asarray(_I_ARR), jnp.asarray(_C_ARR), adj4, s2, part, b2r)

    return out


# speedup vs baseline: 10.2274x; 10.2274x over previous
import numpy as np

import jax
import jax.numpy as jnp
from jax.experimental import pallas as pl
from jax.experimental.pallas import tpu as pltpu

N = 10000
NFEAT = 128
NHID = 64
NCLASS = 16
ROWS = 400
NBI = N // ROWS
CW = 1024
NBC = 10
NPAD = NBC * CW


def _pass1_kernel(x_ref, adj_ref, w1_ref, b1_ref, w2_ref,
                  part_ref, s2out_ref, s1_ref, s2s_ref):
    i = pl.program_id(0)

    @pl.when(i == 0)
    def _():
        s1_ref[...] = jnp.dot(x_ref[...], w1_ref[...],
                              preferred_element_type=jnp.float32)
        s2s_ref[...] = jnp.zeros_like(s2s_ref)

    adj_blk = adj_ref[...]
    h = jnp.dot(adj_blk, s1_ref[...], preferred_element_type=jnp.float32)
    h = jnp.maximum(h + b1_ref[...], 0.0)
    bound = (i * ROWS) // CW * CW
    row_ids = jax.lax.broadcasted_iota(jnp.int32, (N, NCLASS), 0)
    s2_masked = jnp.where(row_ids < bound, s2s_ref[...], 0.0)
    part_ref[...] = jnp.dot(adj_blk, s2_masked,
                            preferred_element_type=jnp.float32)
    s2_i = jnp.dot(h, w2_ref[...], preferred_element_type=jnp.float32)
    s2s_ref[pl.ds(i * ROWS, ROWS), :] = s2_i
    s2out_ref[...] = s2_i


def _pass2_kernel(i_ref, c_ref, adj_ref, s2_ref, part_ref, b2_ref, o_ref):
    t = pl.program_id(0)
    i = i_ref[t]
    c = c_ref[t]
    col_ids = c * CW + jax.lax.broadcasted_iota(jnp.int32, (ROWS, CW), 1)
    blk = jnp.where(col_ids < N, adj_ref[...], 0.0)
    contrib = jnp.dot(blk, s2_ref[...], preferred_element_type=jnp.float32)
    first = c == (i * ROWS) // CW

    @pl.when(first)
    def _():
        o_ref[...] = part_ref[...] + contrib

    @pl.when(jnp.logical_not(first))
    def _():
        o_ref[...] += contrib

    @pl.when(c == NBC - 1)
    def _():
        z = o_ref[...] + b2_ref[...]
        m = jnp.max(z, axis=1, keepdims=True)
        shifted = z - m
        lse = jnp.log(jnp.sum(jnp.exp(shifted), axis=1, keepdims=True))
        o_ref[...] = shifted - lse


_PAIRS = [(i, c) for i in range(NBI) for c in range((i * ROWS) // CW, NBC)]
_I_ARR = np.array([p[0] for p in _PAIRS], dtype=np.int32)
_C_ARR = np.array([p[1] for p in _PAIRS], dtype=np.int32)
_T = len(_PAIRS)


@jax.jit
def kernel(x, adj, W1, b1, W2, b2):
    b1r = b1.reshape(1, NHID)
    b2r = b2.reshape(1, NCLASS)

    part, s2 = pl.pallas_call(
        _pass1_kernel,
        grid=(NBI,),
        in_specs=[
            pl.BlockSpec((N, NFEAT), lambda i: (0, 0)),
            pl.BlockSpec((ROWS, N), lambda i: (i, 0)),
            pl.BlockSpec((NFEAT, NHID), lambda i: (0, 0)),
            pl.BlockSpec((1, NHID), lambda i: (0, 0)),
            pl.BlockSpec((NHID, NCLASS), lambda i: (0, 0)),
        ],
        out_specs=[
            pl.BlockSpec((ROWS, NCLASS), lambda i: (i, 0)),
            pl.BlockSpec((ROWS, NCLASS), lambda i: (i, 0)),
        ],
        out_shape=[
            jax.ShapeDtypeStruct((N, NCLASS), jnp.float32),
            jax.ShapeDtypeStruct((N, NCLASS), jnp.float32),
        ],
        scratch_shapes=[
            pltpu.VMEM((N, NHID), jnp.float32),
            pltpu.VMEM((N, NCLASS), jnp.float32),
        ],
        compiler_params=pltpu.CompilerParams(
            dimension_semantics=("arbitrary",),
        ),
    )(x, adj, W1, b1r, W2)

    s2p = jnp.concatenate(
        [s2, jnp.zeros((NPAD - N, NCLASS), jnp.float32)], axis=0)

    out = pl.pallas_call(
        _pass2_kernel,
        grid_spec=pltpu.PrefetchScalarGridSpec(
            num_scalar_prefetch=2,
            grid=(_T,),
            in_specs=[
                pl.BlockSpec((ROWS, CW),
                             lambda t, i_ref, c_ref: (i_ref[t], c_ref[t])),
                pl.BlockSpec((CW, NCLASS),
                             lambda t, i_ref, c_ref: (c_ref[t], 0)),
                pl.BlockSpec((ROWS, NCLASS),
                             lambda t, i_ref, c_ref: (i_ref[t], 0)),
                pl.BlockSpec((1, NCLASS),
                             lambda t, i_ref, c_ref: (0, 0)),
            ],
            out_specs=pl.BlockSpec(
                (ROWS, NCLASS), lambda t, i_ref, c_ref: (i_ref[t], 0)),
        ),
        out_shape=jax.ShapeDtypeStruct((N, NCLASS), jnp.float32),
        compiler_params=pltpu.CompilerParams(
            dimension_semantics=("arbitrary",),
        ),
    )(jnp.asarray(_I_ARR), jnp.asarray(_C_ARR), adj, s2p, part, b2r)

    return out


# speedup vs baseline: 13.2264x; 1.2932x over previous
import numpy as np

import jax
import jax.numpy as jnp
from jax.experimental import pallas as pl
from jax.experimental.pallas import tpu as pltpu

N = 10000
NFEAT = 128
NHID = 64
NCLASS = 16
ROWS = 400
NBI = N // ROWS
RW2 = 800
CW2 = 2048
NG = 13
NBC2 = 5
NPAD = NBC2 * CW2


def _cmin_group(g):
    return (RW2 * g) // CW2


def _pass1_kernel(x_ref, adj_ref, w1_ref, b1_ref, w2_ref,
                  part_ref, s2out_ref, s1_ref, s2s_ref):
    i = pl.program_id(0)

    @pl.when(i == 0)
    def _():
        s1_ref[...] = jnp.dot(x_ref[...], w1_ref[...],
                              preferred_element_type=jnp.float32)
        s2s_ref[...] = jnp.zeros_like(s2s_ref)

    h = jnp.dot(adj_ref[...], s1_ref[...],
                preferred_element_type=jnp.float32)
    h = jnp.maximum(h + b1_ref[...], 0.0)

    cmin = (RW2 * (i // 2)) // CW2
    part_ref[...] = jnp.zeros_like(part_ref)
    for k in range(4):
        @pl.when(k < cmin)
        def _():
            part_ref[...] += jnp.dot(
                adj_ref[:, k * CW2:(k + 1) * CW2],
                s2s_ref[k * CW2:(k + 1) * CW2, :],
                preferred_element_type=jnp.float32)

    s2_i = jnp.dot(h, w2_ref[...], preferred_element_type=jnp.float32)
    s2s_ref[pl.ds(i * ROWS, ROWS), :] = s2_i
    s2out_ref[...] = s2_i


def _pass2_kernel(g_ref, c_ref, adj_ref, s2_ref, part_ref, b2_ref, o_ref):
    t = pl.program_id(0)
    g = g_ref[t]
    c = c_ref[t]
    first = c == (RW2 * g) // CW2

    @pl.when(c != NBC2 - 1)
    def _():
        contrib = jnp.dot(adj_ref[...], s2_ref[...],
                          preferred_element_type=jnp.float32)
        base = jnp.where(first, part_ref[...], o_ref[...])
        o_ref[...] = base + contrib

    @pl.when(c == NBC2 - 1)
    def _():
        col_ids = jax.lax.broadcasted_iota(jnp.int32, (RW2, CW2), 1)
        blk = jnp.where(col_ids < N - (NBC2 - 1) * CW2, adj_ref[...], 0.0)
        contrib = jnp.dot(blk, s2_ref[...],
                          preferred_element_type=jnp.float32)
        base = jnp.where(first, part_ref[...], o_ref[...])
        z = base + contrib + b2_ref[...]
        m = jnp.max(z, axis=1, keepdims=True)
        shifted = z - m
        lse = jnp.log(jnp.sum(jnp.exp(shifted), axis=1, keepdims=True))
        o_ref[...] = shifted - lse


_PAIRS = [(g, c) for g in range(NG) for c in range(_cmin_group(g), NBC2)]
_G_ARR = np.array([p[0] for p in _PAIRS], dtype=np.int32)
_C_ARR = np.array([p[1] for p in _PAIRS], dtype=np.int32)
_T = len(_PAIRS)


@jax.jit
def kernel(x, adj, W1, b1, W2, b2):
    b1r = b1.reshape(1, NHID)
    b2r = b2.reshape(1, NCLASS)

    part, s2 = pl.pallas_call(
        _pass1_kernel,
        grid=(NBI,),
        in_specs=[
            pl.BlockSpec((N, NFEAT), lambda i: (0, 0)),
            pl.BlockSpec((ROWS, N), lambda i: (i, 0)),
            pl.BlockSpec((NFEAT, NHID), lambda i: (0, 0)),
            pl.BlockSpec((1, NHID), lambda i: (0, 0)),
            pl.BlockSpec((NHID, NCLASS), lambda i: (0, 0)),
        ],
        out_specs=[
            pl.BlockSpec((ROWS, NCLASS), lambda i: (i, 0)),
            pl.BlockSpec((ROWS, NCLASS), lambda i: (i, 0)),
        ],
        out_shape=[
            jax.ShapeDtypeStruct((N, NCLASS), jnp.float32),
            jax.ShapeDtypeStruct((N, NCLASS), jnp.float32),
        ],
        scratch_shapes=[
            pltpu.VMEM((N, NHID), jnp.float32),
            pltpu.VMEM((N, NCLASS), jnp.float32),
        ],
        compiler_params=pltpu.CompilerParams(
            dimension_semantics=("arbitrary",),
        ),
    )(x, adj, W1, b1r, W2)

    s2p = jnp.concatenate(
        [s2, jnp.zeros((NPAD - N, NCLASS), jnp.float32)], axis=0)

    out = pl.pallas_call(
        _pass2_kernel,
        grid_spec=pltpu.PrefetchScalarGridSpec(
            num_scalar_prefetch=2,
            grid=(_T,),
            in_specs=[
                pl.BlockSpec((RW2, CW2),
                             lambda t, g_ref, c_ref: (g_ref[t], c_ref[t])),
                pl.BlockSpec((CW2, NCLASS),
                             lambda t, g_ref, c_ref: (c_ref[t], 0)),
                pl.BlockSpec((RW2, NCLASS),
                             lambda t, g_ref, c_ref: (g_ref[t], 0)),
                pl.BlockSpec((1, NCLASS),
                             lambda t, g_ref, c_ref: (0, 0)),
            ],
            out_specs=pl.BlockSpec(
                (RW2, NCLASS), lambda t, g_ref, c_ref: (g_ref[t], 0)),
        ),
        out_shape=jax.ShapeDtypeStruct((N, NCLASS), jnp.float32),
        compiler_params=pltpu.CompilerParams(
            dimension_semantics=("arbitrary",),
        ),
    )(jnp.asarray(_G_ARR), jnp.asarray(_C_ARR), adj, s2p, part, b2r)

    return out


# speedup vs baseline: 13.2297x; 1.0002x over previous
import numpy as np

import jax
import jax.numpy as jnp
from jax.experimental import pallas as pl
from jax.experimental.pallas import tpu as pltpu

N = 10000
NFEAT = 128
NHID = 64
NCLASS = 16
ROWS = 400
NBI = N // ROWS
RW2 = 800
CW2 = 2048
NG = 13
NBC2 = 5
NPAD = NBC2 * CW2


def _cmin_group(g):
    return (RW2 * g) // CW2


def _pass1_kernel(x_ref, adj_ref, w1_ref, b1_ref, w2_ref,
                  part_ref, s2out_ref, s1_ref, s2s_ref):
    i = pl.program_id(0)

    @pl.when(i == 0)
    def _():
        s1_ref[...] = jnp.dot(x_ref[...], w1_ref[...],
                              preferred_element_type=jnp.float32
                              ).astype(jnp.bfloat16)
        s2s_ref[...] = jnp.zeros_like(s2s_ref)

    abf = adj_ref[...].astype(jnp.bfloat16)
    h = jnp.dot(abf, s1_ref[...], preferred_element_type=jnp.float32)
    h = jnp.maximum(h + b1_ref[...], 0.0)

    cmin = (RW2 * (i // 2)) // CW2
    part_ref[...] = jnp.zeros_like(part_ref)
    for k in range(4):
        @pl.when(k < cmin)
        def _():
            part_ref[...] += jnp.dot(
                abf[:, k * CW2:(k + 1) * CW2],
                s2s_ref[k * CW2:(k + 1) * CW2, :],
                preferred_element_type=jnp.float32)

    s2_i = jnp.dot(h, w2_ref[...], preferred_element_type=jnp.float32)
    s2s_ref[pl.ds(i * ROWS, ROWS), :] = s2_i.astype(jnp.bfloat16)
    s2out_ref[...] = s2_i


def _pass2_kernel(g_ref, c_ref, adj_ref, s2_ref, part_ref, b2_ref, o_ref):
    t = pl.program_id(0)
    g = g_ref[t]
    c = c_ref[t]
    first = c == (RW2 * g) // CW2

    @pl.when(c != NBC2 - 1)
    def _():
        contrib = jnp.dot(adj_ref[...], s2_ref[...],
                          preferred_element_type=jnp.float32)
        base = jnp.where(first, part_ref[...], o_ref[...])
        o_ref[...] = base + contrib

    @pl.when(c == NBC2 - 1)
    def _():
        col_ids = jax.lax.broadcasted_iota(jnp.int32, (RW2, CW2), 1)
        blk = jnp.where(col_ids < N - (NBC2 - 1) * CW2, adj_ref[...], 0.0)
        contrib = jnp.dot(blk, s2_ref[...],
                          preferred_element_type=jnp.float32)
        base = jnp.where(first, part_ref[...], o_ref[...])
        z = base + contrib + b2_ref[...]
        m = jnp.max(z, axis=1, keepdims=True)
        shifted = z - m
        lse = jnp.log(jnp.sum(jnp.exp(shifted), axis=1, keepdims=True))
        o_ref[...] = shifted - lse


_PAIRS = [(g, c) for g in range(NG) for c in range(_cmin_group(g), NBC2)]
_G_ARR = np.array([p[0] for p in _PAIRS], dtype=np.int32)
_C_ARR = np.array([p[1] for p in _PAIRS], dtype=np.int32)
_T = len(_PAIRS)


@jax.jit
def kernel(x, adj, W1, b1, W2, b2):
    b1r = b1.reshape(1, NHID)
    b2r = b2.reshape(1, NCLASS)

    part, s2 = pl.pallas_call(
        _pass1_kernel,
        grid=(NBI,),
        in_specs=[
            pl.BlockSpec((N, NFEAT), lambda i: (0, 0)),
            pl.BlockSpec((ROWS, N), lambda i: (i, 0)),
            pl.BlockSpec((NFEAT, NHID), lambda i: (0, 0)),
            pl.BlockSpec((1, NHID), lambda i: (0, 0)),
            pl.BlockSpec((NHID, NCLASS), lambda i: (0, 0)),
        ],
        out_specs=[
            pl.BlockSpec((ROWS, NCLASS), lambda i: (i, 0)),
            pl.BlockSpec((ROWS, NCLASS), lambda i: (i, 0)),
        ],
        out_shape=[
            jax.ShapeDtypeStruct((N, NCLASS), jnp.float32),
            jax.ShapeDtypeStruct((N, NCLASS), jnp.float32),
        ],
        scratch_shapes=[
            pltpu.VMEM((N, NHID), jnp.bfloat16),
            pltpu.VMEM((N, NCLASS), jnp.bfloat16),
        ],
        compiler_params=pltpu.CompilerParams(
            dimension_semantics=("arbitrary",),
        ),
    )(x, adj, W1, b1r, W2)

    s2p = jnp.concatenate(
        [s2, jnp.zeros((NPAD - N, NCLASS), jnp.float32)], axis=0)

    out = pl.pallas_call(
        _pass2_kernel,
        grid_spec=pltpu.PrefetchScalarGridSpec(
            num_scalar_prefetch=2,
            grid=(_T,),
            in_specs=[
                pl.BlockSpec((RW2, CW2),
                             lambda t, g_ref, c_ref: (g_ref[t], c_ref[t])),
                pl.BlockSpec((CW2, NCLASS),
                             lambda t, g_ref, c_ref: (c_ref[t], 0)),
                pl.BlockSpec((RW2, NCLASS),
                             lambda t, g_ref, c_ref: (g_ref[t], 0)),
                pl.BlockSpec((1, NCLASS),
                             lambda t, g_ref, c_ref: (0, 0)),
            ],
            out_specs=pl.BlockSpec(
                (RW2, NCLASS), lambda t, g_ref, c_ref: (g_ref[t], 0)),
        ),
        out_shape=jax.ShapeDtypeStruct((N, NCLASS), jnp.float32),
        compiler_params=pltpu.CompilerParams(
            dimension_semantics=("arbitrary",),
        ),
    )(jnp.asarray(_G_ARR), jnp.asarray(_C_ARR), adj, s2p, part, b2r)

    return out


# speedup vs baseline: 13.5319x; 1.0228x over previous
import numpy as np

import jax
import jax.numpy as jnp
from jax.experimental import pallas as pl
from jax.experimental.pallas import tpu as pltpu

N = 10000
NFEAT = 128
NHID = 64
NCLASS = 16
ROWS = 400
NBI = N // ROWS
RW2 = 800
CW2 = 2048
NG = 13
NBC2 = 5
NPAD = NBC2 * CW2


def _cmin_group(g):
    return (RW2 * g) // CW2


_COPY_AT = []
for _m in range(4):
    _COPY_AT.append(min(
        i for i in range(NBI) if (RW2 * (i // 2)) // CW2 == _m + 1))


def _pass1_kernel(x_ref, adj_ref, w1_ref, b1_ref, w2_ref,
                  part_ref, s2out_ref, s1s2_ref, s2f_ref):
    i = pl.program_id(0)

    @pl.when(i == 0)
    def _():
        s1s2_ref[:, :NHID] = jnp.dot(
            x_ref[...], w1_ref[...],
            preferred_element_type=jnp.float32).astype(jnp.bfloat16)
        s1s2_ref[:, NHID:] = jnp.zeros((N, NCLASS), jnp.bfloat16)

    for _m, _strip in enumerate(_COPY_AT):
        @pl.when(i == _strip)
        def _():
            s1s2_ref[_m * CW2:(_m + 1) * CW2, NHID:] = \
                s2f_ref[_m * CW2:(_m + 1) * CW2, :]

    abf = adj_ref[...].astype(jnp.bfloat16)
    hp = jnp.dot(abf, s1s2_ref[...], preferred_element_type=jnp.float32)
    h = jnp.maximum(hp[:, :NHID] + b1_ref[...], 0.0)
    part_ref[...] = hp[:, NHID:]
    s2_i = jnp.dot(h, w2_ref[...], preferred_element_type=jnp.float32)
    s2f_ref[pl.ds(i * ROWS, ROWS), :] = s2_i.astype(jnp.bfloat16)
    s2out_ref[...] = s2_i


def _pass2_kernel(g_ref, c_ref, adj_ref, s2_ref, part_ref, b2_ref, o_ref):
    t = pl.program_id(0)
    g = g_ref[t]
    c = c_ref[t]
    first = c == (RW2 * g) // CW2

    @pl.when(c != NBC2 - 1)
    def _():
        contrib = jnp.dot(adj_ref[...], s2_ref[...],
                          preferred_element_type=jnp.float32)
        base = jnp.where(first, part_ref[...], o_ref[...])
        o_ref[...] = base + contrib

    @pl.when(c == NBC2 - 1)
    def _():
        col_ids = jax.lax.broadcasted_iota(jnp.int32, (RW2, CW2), 1)
        blk = jnp.where(col_ids < N - (NBC2 - 1) * CW2, adj_ref[...], 0.0)
        contrib = jnp.dot(blk, s2_ref[...],
                          preferred_element_type=jnp.float32)
        base = jnp.where(first, part_ref[...], o_ref[...])
        z = base + contrib + b2_ref[...]
        m = jnp.max(z, axis=1, keepdims=True)
        shifted = z - m
        lse = jnp.log(jnp.sum(jnp.exp(shifted), axis=1, keepdims=True))
        o_ref[...] = shifted - lse


_PAIRS = [(g, c) for g in range(NG) for c in range(_cmin_group(g), NBC2)]
_G_ARR = np.array([p[0] for p in _PAIRS], dtype=np.int32)
_C_ARR = np.array([p[1] for p in _PAIRS], dtype=np.int32)
_T = len(_PAIRS)


@jax.jit
def kernel(x, adj, W1, b1, W2, b2):
    b1r = b1.reshape(1, NHID)
    b2r = b2.reshape(1, NCLASS)

    part, s2 = pl.pallas_call(
        _pass1_kernel,
        grid=(NBI,),
        in_specs=[
            pl.BlockSpec((N, NFEAT), lambda i: (0, 0)),
            pl.BlockSpec((ROWS, N), lambda i: (i, 0)),
            pl.BlockSpec((NFEAT, NHID), lambda i: (0, 0)),
            pl.BlockSpec((1, NHID), lambda i: (0, 0)),
            pl.BlockSpec((NHID, NCLASS), lambda i: (0, 0)),
        ],
        out_specs=[
            pl.BlockSpec((ROWS, NCLASS), lambda i: (i, 0)),
            pl.BlockSpec((ROWS, NCLASS), lambda i: (i, 0)),
        ],
        out_shape=[
            jax.ShapeDtypeStruct((N, NCLASS), jnp.float32),
            jax.ShapeDtypeStruct((N, NCLASS), jnp.float32),
        ],
        scratch_shapes=[
            pltpu.VMEM((N, NHID + NCLASS), jnp.bfloat16),
            pltpu.VMEM((N, NCLASS), jnp.bfloat16),
        ],
        compiler_params=pltpu.CompilerParams(
            dimension_semantics=("arbitrary",),
        ),
    )(x, adj, W1, b1r, W2)

    s2p = jnp.concatenate(
        [s2, jnp.zeros((NPAD - N, NCLASS), jnp.float32)], axis=0)

    out = pl.pallas_call(
        _pass2_kernel,
        grid_spec=pltpu.PrefetchScalarGridSpec(
            num_scalar_prefetch=2,
            grid=(_T,),
            in_specs=[
                pl.BlockSpec((RW2, CW2),
                             lambda t, g_ref, c_ref: (g_ref[t], c_ref[t])),
                pl.BlockSpec((CW2, NCLASS),
                             lambda t, g_ref, c_ref: (c_ref[t], 0)),
                pl.BlockSpec((RW2, NCLASS),
                             lambda t, g_ref, c_ref: (g_ref[t], 0)),
                pl.BlockSpec((1, NCLASS),
                             lambda t, g_ref, c_ref: (0, 0)),
            ],
            out_specs=pl.BlockSpec(
                (RW2, NCLASS), lambda t, g_ref, c_ref: (g_ref[t], 0)),
        ),
        out_shape=jax.ShapeDtypeStruct((N, NCLASS), jnp.float32),
        compiler_params=pltpu.CompilerParams(
            dimension_semantics=("arbitrary",),
        ),
    )(jnp.asarray(_G_ARR), jnp.asarray(_C_ARR), adj, s2p, part, b2r)

    return out


# speedup vs baseline: 13.7813x; 1.0184x over previous
import numpy as np

import jax
import jax.numpy as jnp
from jax.experimental import pallas as pl
from jax.experimental.pallas import tpu as pltpu

N = 10000
NFEAT = 128
NHID = 64
NCLASS = 16
NW = NHID + NCLASS

ROWS = 200
NBI = N // ROWS
RW2 = 800
CW2 = 2048
NG = 13
NBC2 = 5
EDGE = N - (NBC2 - 1) * CW2
SGRP = 3
SROWS = SGRP * RW2


def _cmin_group(g):
    return (RW2 * g) // CW2


_COPY_AT = []
for _m in range(NBC2 - 1):
    _COPY_AT.append(min(
        i for i in range(NBI) if _cmin_group(i // (RW2 // ROWS)) == _m + 1))


def _xw_kernel(x_ref, w_ref, o_ref):
    o_ref[...] = jnp.dot(x_ref[...], w_ref[...],
                         preferred_element_type=jnp.float32)


def _main_kernel(ia_ref, gb_ref, cb_ref, ph_ref, gg_ref, cc_ref, og_ref,
                 s1_ref, adja_ref, adjb_ref, w2_ref, b1_ref, b2_ref,
                 o_ref, s1s2_ref, s2v_ref, part_ref, stash_ref):
    t = pl.program_id(0)
    ph = ph_ref[t]
    g = gg_ref[t]
    c = cc_ref[t]

    @pl.when(t == 0)
    def _():
        s1s2_ref[...] = jnp.zeros_like(s1s2_ref)
        s1s2_ref[0:N, :NHID] = s1_ref[...].astype(jnp.bfloat16)
        s2v_ref[N:, :] = jnp.zeros_like(s2v_ref[N:, :])

    for _m, _strip in enumerate(_COPY_AT):
        @pl.when(jnp.logical_and(ph == 0, g == _strip))
        def _():
            s1s2_ref[_m * CW2:(_m + 1) * CW2, NHID:] = \
                s2v_ref[_m * CW2:(_m + 1) * CW2, :].astype(jnp.bfloat16)

    @pl.when(ph == 0)
    def _():
        abf = adja_ref[...].astype(jnp.bfloat16)
        hp = jnp.dot(abf, s1s2_ref[0:N, :],
                     preferred_element_type=jnp.float32)
        h = jnp.maximum(hp[:, :NHID] + b1_ref[...], 0.0)
        s2_i = jnp.dot(h, w2_ref[...], preferred_element_type=jnp.float32)
        s2v_ref[pl.ds(g * ROWS, ROWS), :] = s2_i
        part_ref[pl.ds(g * ROWS, ROWS), :] = hp[:, NHID:]

        @pl.when(g < SROWS // ROWS)
        def _():
            stash_ref[pl.ds(g * ROWS, ROWS), :] = abf[:, (NBC2 - 1) * CW2:N]

    first = c == (RW2 * g) // CW2

    def _accum_and_finish(contrib, last):
        base = jnp.where(first, part_ref[pl.ds(g * RW2, RW2), :],
                         o_ref[...])
        if not last:
            o_ref[...] = base + contrib
        else:
            z = base + contrib + b2_ref[...]
            m = jnp.max(z, axis=1, keepdims=True)
            shifted = z - m
            lse = jnp.log(jnp.sum(jnp.exp(shifted), axis=1, keepdims=True))
            o_ref[...] = shifted - lse

    @pl.when(jnp.logical_and(ph == 1, c != NBC2 - 1))
    def _():
        contrib = jnp.dot(adjb_ref[...],
                          s2v_ref[pl.ds(c * CW2, CW2), :],
                          preferred_element_type=jnp.float32)
        _accum_and_finish(contrib, last=False)

    edge = jnp.logical_and(ph == 1, c == NBC2 - 1)

    @pl.when(jnp.logical_and(edge, g < SGRP))
    def _():
        ablk = stash_ref[pl.ds(g * RW2, RW2), :]
        s2c = s2v_ref[(NBC2 - 1) * CW2:N, :].astype(jnp.bfloat16)
        contrib = jnp.dot(ablk, s2c, preferred_element_type=jnp.float32)
        _accum_and_finish(contrib, last=True)

    @pl.when(jnp.logical_and(edge, g >= SGRP))
    def _():
        col_ids = jax.lax.broadcasted_iota(jnp.int32, (RW2, CW2), 1)
        blk = jnp.where(col_ids < EDGE, adjb_ref[...], 0.0)
        contrib = jnp.dot(blk,
                          s2v_ref[(NBC2 - 1) * CW2:(NBC2 - 1) * CW2 + CW2, :],
                          preferred_element_type=jnp.float32)
        _accum_and_finish(contrib, last=True)


def _schedule():
    ia, gb, cb, ph, gg, cc, og = [], [], [], [], [], [], []
    for i in range(NBI):
        ia.append(i); gb.append(0); cb.append(0)
        ph.append(0); gg.append(i); cc.append(0); og.append(0)
    last_b = (0, 0)
    for g in range(NG):
        for c in range(_cmin_group(g), NBC2):
            if c != NBC2 - 1 or g >= SGRP:
                last_b = (g, c)
            ia.append(NBI - 1)
            gb.append(last_b[0]); cb.append(last_b[1])
            ph.append(1); gg.append(g); cc.append(c); og.append(g)
    mk = lambda v: np.asarray(v, dtype=np.int32)
    return tuple(mk(v) for v in (ia, gb, cb, ph, gg, cc, og))


_IA, _GB, _CB, _PH, _GG, _CC, _OG = _schedule()
_T = len(_IA)


@jax.jit
def kernel(x, adj, W1, b1, W2, b2):
    b1r = b1.reshape(1, NHID)
    b2r = b2.reshape(1, NCLASS)

    s1 = pl.pallas_call(
        _xw_kernel,
        out_shape=jax.ShapeDtypeStruct((N, NHID), jnp.float32),
    )(x, W1)

    out = pl.pallas_call(
        _main_kernel,
        grid_spec=pltpu.PrefetchScalarGridSpec(
            num_scalar_prefetch=7,
            grid=(_T,),
            in_specs=[
                pl.BlockSpec((N, NHID), lambda t, *s: (0, 0)),
                pl.BlockSpec((ROWS, N), lambda t, *s: (s[0][t], 0)),
                pl.BlockSpec((RW2, CW2), lambda t, *s: (s[1][t], s[2][t])),
                pl.BlockSpec((NHID, NCLASS), lambda t, *s: (0, 0)),
                pl.BlockSpec((1, NHID), lambda t, *s: (0, 0)),
                pl.BlockSpec((1, NCLASS), lambda t, *s: (0, 0)),
            ],
            out_specs=pl.BlockSpec(
                (RW2, NCLASS), lambda t, *s: (s[6][t], 0)),
            scratch_shapes=[
                pltpu.VMEM((N, NW), jnp.bfloat16),
                pltpu.VMEM((NG * RW2 + 32, NCLASS), jnp.float32),
                pltpu.VMEM((NG * RW2 + 32, NCLASS), jnp.float32),
                pltpu.VMEM((SROWS, EDGE), jnp.bfloat16),
            ],
        ),
        out_shape=jax.ShapeDtypeStruct((N, NCLASS), jnp.float32),
        compiler_params=pltpu.CompilerParams(
            dimension_semantics=("arbitrary",),
        ),
    )(jnp.asarray(_IA), jnp.asarray(_GB), jnp.asarray(_CB),
      jnp.asarray(_PH), jnp.asarray(_GG), jnp.asarray(_CC),
      jnp.asarray(_OG), s1, adj, adj, W2, b1r, b2r)

    return out


# speedup vs baseline: 14.0981x; 1.0230x over previous
import numpy as np

import jax
import jax.numpy as jnp
from jax.experimental import pallas as pl
from jax.experimental.pallas import tpu as pltpu

N = 10000
NFEAT = 128
NHID = 64
NCLASS = 16
NW = NHID + NCLASS

ROWS = 200
NBI = N // ROWS
RW2 = 800
CW2 = 2048
NG = 13
NBC2 = 5
EDGE = N - (NBC2 - 1) * CW2
SGRP = 3
SROWS = SGRP * RW2


def _cmin_group(g):
    return (RW2 * g) // CW2


_COPY_AT = []
for _m in range(NBC2 - 1):
    _COPY_AT.append(min(
        i for i in range(NBI) if _cmin_group(i // (RW2 // ROWS)) == _m + 1))


def _main_kernel(ia_ref, gb_ref, cb_ref, ph_ref, gg_ref, cc_ref, og_ref,
                 x_ref, adja_ref, adjb_ref, w1_ref, w2_ref, b1_ref, b2_ref,
                 o_ref, s1s2_ref, s2v_ref, part_ref, stash_ref):
    t = pl.program_id(0)
    ph = ph_ref[t]
    g = gg_ref[t]
    c = cc_ref[t]

    @pl.when(t == 0)
    def _():
        s1s2_ref[...] = jnp.zeros_like(s1s2_ref)
        s1s2_ref[0:N, :NHID] = jnp.dot(
            x_ref[...], w1_ref[...],
            preferred_element_type=jnp.float32).astype(jnp.bfloat16)
        s2v_ref[N:, :] = jnp.zeros_like(s2v_ref[N:, :])

    for _m, _strip in enumerate(_COPY_AT):
        @pl.when(jnp.logical_and(ph == 0, g == _strip))
        def _():
            s1s2_ref[_m * CW2:(_m + 1) * CW2, NHID:] = \
                s2v_ref[_m * CW2:(_m + 1) * CW2, :].astype(jnp.bfloat16)

    @pl.when(ph == 0)
    def _():
        abf = adja_ref[...].astype(jnp.bfloat16)
        hp = jnp.dot(abf, s1s2_ref[0:N, :],
                     preferred_element_type=jnp.float32)
        h = jnp.maximum(hp[:, :NHID] + b1_ref[...], 0.0)
        s2_i = jnp.dot(h, w2_ref[...], preferred_element_type=jnp.float32)
        s2v_ref[pl.ds(g * ROWS, ROWS), :] = s2_i
        part_ref[pl.ds(g * ROWS, ROWS), :] = hp[:, NHID:]

        @pl.when(g < SROWS // ROWS)
        def _():
            stash_ref[pl.ds(g * ROWS, ROWS), :] = abf[:, (NBC2 - 1) * CW2:N]

    first = c == (RW2 * g) // CW2

    def _accum_and_finish(contrib, last):
        base = jnp.where(first, part_ref[pl.ds(g * RW2, RW2), :],
                         o_ref[...])
        if not last:
            o_ref[...] = base + contrib
        else:
            z = base + contrib + b2_ref[...]
            m = jnp.max(z, axis=1, keepdims=True)
            shifted = z - m
            lse = jnp.log(jnp.sum(jnp.exp(shifted), axis=1, keepdims=True))
            o_ref[...] = shifted - lse

    @pl.when(jnp.logical_and(ph == 1, c != NBC2 - 1))
    def _():
        contrib = jnp.dot(adjb_ref[...],
                          s2v_ref[pl.ds(c * CW2, CW2), :],
                          preferred_element_type=jnp.float32)
        _accum_and_finish(contrib, last=False)

    edge = jnp.logical_and(ph == 1, c == NBC2 - 1)

    @pl.when(jnp.logical_and(edge, g < SGRP))
    def _():
        ablk = stash_ref[pl.ds(g * RW2, RW2), :]
        s2c = s2v_ref[(NBC2 - 1) * CW2:N, :].astype(jnp.bfloat16)
        contrib = jnp.dot(ablk, s2c, preferred_element_type=jnp.float32)
        _accum_and_finish(contrib, last=True)

    @pl.when(jnp.logical_and(edge, g >= SGRP))
    def _():
        col_ids = jax.lax.broadcasted_iota(jnp.int32, (RW2, CW2), 1)
        blk = jnp.where(col_ids < EDGE, adjb_ref[...], 0.0)
        contrib = jnp.dot(blk,
                          s2v_ref[(NBC2 - 1) * CW2:(NBC2 - 1) * CW2 + CW2, :],
                          preferred_element_type=jnp.float32)
        _accum_and_finish(contrib, last=True)


def _schedule():
    ia, gb, cb, ph, gg, cc, og = [], [], [], [], [], [], []
    for i in range(NBI):
        ia.append(i); gb.append(0); cb.append(0)
        ph.append(0); gg.append(i); cc.append(0); og.append(0)
    last_b = (0, 0)
    for g in range(NG):
        for c in range(_cmin_group(g), NBC2):
            if c != NBC2 - 1 or g >= SGRP:
                last_b = (g, c)
            ia.append(NBI - 1)
            gb.append(last_b[0]); cb.append(last_b[1])
            ph.append(1); gg.append(g); cc.append(c); og.append(g)
    mk = lambda v: np.asarray(v, dtype=np.int32)
    return tuple(mk(v) for v in (ia, gb, cb, ph, gg, cc, og))


_IA, _GB, _CB, _PH, _GG, _CC, _OG = _schedule()
_T = len(_IA)


@jax.jit
def kernel(x, adj, W1, b1, W2, b2):
    b1r = b1.reshape(1, NHID)
    b2r = b2.reshape(1, NCLASS)

    out = pl.pallas_call(
        _main_kernel,
        grid_spec=pltpu.PrefetchScalarGridSpec(
            num_scalar_prefetch=7,
            grid=(_T,),
            in_specs=[
                pl.BlockSpec((N, NFEAT), lambda t, *s: (0, 0)),
                pl.BlockSpec((ROWS, N), lambda t, *s: (s[0][t], 0)),
                pl.BlockSpec((RW2, CW2), lambda t, *s: (s[1][t], s[2][t])),
                pl.BlockSpec((NFEAT, NHID), lambda t, *s: (0, 0)),
                pl.BlockSpec((NHID, NCLASS), lambda t, *s: (0, 0)),
                pl.BlockSpec((1, NHID), lambda t, *s: (0, 0)),
                pl.BlockSpec((1, NCLASS), lambda t, *s: (0, 0)),
            ],
            out_specs=pl.BlockSpec(
                (RW2, NCLASS), lambda t, *s: (s[6][t], 0)),
            scratch_shapes=[
                pltpu.VMEM((N, NW), jnp.bfloat16),
                pltpu.VMEM((NG * RW2 + 32, NCLASS), jnp.float32),
                pltpu.VMEM((NG * RW2 + 32, NCLASS), jnp.float32),
                pltpu.VMEM((SROWS, EDGE), jnp.bfloat16),
            ],
        ),
        out_shape=jax.ShapeDtypeStruct((N, NCLASS), jnp.float32),
        compiler_params=pltpu.CompilerParams(
            dimension_semantics=("arbitrary",),
        ),
    )(jnp.asarray(_IA), jnp.asarray(_GB), jnp.asarray(_CB),
      jnp.asarray(_PH), jnp.asarray(_GG), jnp.asarray(_CC),
      jnp.asarray(_OG), x, adj, adj, W1, W2, b1r, b2r)

    return out


# speedup vs baseline: 14.1227x; 1.0017x over previous
import numpy as np

import jax
import jax.numpy as jnp
from jax.experimental import pallas as pl
from jax.experimental.pallas import tpu as pltpu

N = 10000
NFEAT = 128
NHID = 64
NCLASS = 16
NW = NHID + NCLASS

ROWS = 200
NBI = N // ROWS
RW2 = 800
CW2 = 2048
NG = 13
NBC2 = 5
EDGE = N - (NBC2 - 1) * CW2
SGRP = 5
SROWS = SGRP * RW2


def _cmin_group(g):
    return (RW2 * g) // CW2


_COPY_AT = []
for _m in range(NBC2 - 1):
    _COPY_AT.append(min(
        i for i in range(NBI) if _cmin_group(i // (RW2 // ROWS)) == _m + 1))


def _main_kernel(ia_ref, gb_ref, cb_ref, ph_ref, gg_ref, cc_ref, og_ref,
                 x_ref, adja_ref, adjb_ref, w1_ref, w2_ref, b1_ref, b2_ref,
                 o_ref, s1s2_ref, s2v_ref, part_ref, stash_ref):
    t = pl.program_id(0)
    ph = ph_ref[t]
    g = gg_ref[t]
    c = cc_ref[t]

    @pl.when(t == 0)
    def _():
        s1s2_ref[...] = jnp.zeros_like(s1s2_ref)
        s1s2_ref[0:N, :NHID] = jnp.dot(
            x_ref[...], w1_ref[...],
            preferred_element_type=jnp.float32).astype(jnp.bfloat16)
        s2v_ref[N:, :] = jnp.zeros_like(s2v_ref[N:, :])

    for _m, _strip in enumerate(_COPY_AT):
        @pl.when(jnp.logical_and(ph == 0, g == _strip))
        def _():
            s1s2_ref[_m * CW2:(_m + 1) * CW2, NHID:] = \
                s2v_ref[_m * CW2:(_m + 1) * CW2, :]

    @pl.when(ph == 0)
    def _():
        abf = adja_ref[...].astype(jnp.bfloat16)
        hp = jnp.dot(abf, s1s2_ref[0:N, :],
                     preferred_element_type=jnp.float32)
        h = jnp.maximum(hp[:, :NHID] + b1_ref[...], 0.0)
        s2_i = jnp.dot(h, w2_ref[...], preferred_element_type=jnp.float32)
        s2v_ref[pl.ds(g * ROWS, ROWS), :] = s2_i.astype(jnp.bfloat16)
        part_ref[pl.ds(g * ROWS, ROWS), :] = hp[:, NHID:].astype(jnp.bfloat16)

        @pl.when(g < SROWS // ROWS)
        def _():
            stash_ref[pl.ds(g * ROWS, ROWS), :] = abf[:, (NBC2 - 1) * CW2:N]

    first = c == (RW2 * g) // CW2

    def _accum_and_finish(contrib, last):
        base = jnp.where(
            first,
            part_ref[pl.ds(g * RW2, RW2), :].astype(jnp.float32),
            o_ref[...])
        if not last:
            o_ref[...] = base + contrib
        else:
            z = base + contrib + b2_ref[...]
            m = jnp.max(z, axis=1, keepdims=True)
            shifted = z - m
            lse = jnp.log(jnp.sum(jnp.exp(shifted), axis=1, keepdims=True))
            o_ref[...] = shifted - lse

    @pl.when(jnp.logical_and(ph == 1, c != NBC2 - 1))
    def _():
        contrib = jnp.dot(adjb_ref[...],
                          s2v_ref[pl.ds(c * CW2, CW2), :].astype(jnp.float32),
                          preferred_element_type=jnp.float32)
        _accum_and_finish(contrib, last=False)

    edge = jnp.logical_and(ph == 1, c == NBC2 - 1)

    @pl.when(jnp.logical_and(edge, g < SGRP))
    def _():
        ablk = stash_ref[pl.ds(g * RW2, RW2), :]
        s2c = s2v_ref[(NBC2 - 1) * CW2:N, :]
        contrib = jnp.dot(ablk, s2c, preferred_element_type=jnp.float32)
        _accum_and_finish(contrib, last=True)

    @pl.when(jnp.logical_and(edge, g >= SGRP))
    def _():
        col_ids = jax.lax.broadcasted_iota(jnp.int32, (RW2, CW2), 1)
        blk = jnp.where(col_ids < EDGE, adjb_ref[...], 0.0)
        contrib = jnp.dot(
            blk,
            s2v_ref[(NBC2 - 1) * CW2:(NBC2 - 1) * CW2 + CW2,
                    :].astype(jnp.float32),
            preferred_element_type=jnp.float32)
        _accum_and_finish(contrib, last=True)


def _schedule():
    ia, gb, cb, ph, gg, cc, og = [], [], [], [], [], [], []
    for i in range(NBI):
        ia.append(i); gb.append(0); cb.append(0)
        ph.append(0); gg.append(i); cc.append(0); og.append(0)
    last_b = (0, 0)
    for g in range(NG):
        for c in range(_cmin_group(g), NBC2):
            if c != NBC2 - 1 or g >= SGRP:
                last_b = (g, c)
            ia.append(NBI - 1)
            gb.append(last_b[0]); cb.append(last_b[1])
            ph.append(1); gg.append(g); cc.append(c); og.append(g)
    mk = lambda v: np.asarray(v, dtype=np.int32)
    return tuple(mk(v) for v in (ia, gb, cb, ph, gg, cc, og))


_IA, _GB, _CB, _PH, _GG, _CC, _OG = _schedule()
_T = len(_IA)


@jax.jit
def kernel(x, adj, W1, b1, W2, b2):
    b1r = b1.reshape(1, NHID)
    b2r = b2.reshape(1, NCLASS)

    out = pl.pallas_call(
        _main_kernel,
        grid_spec=pltpu.PrefetchScalarGridSpec(
            num_scalar_prefetch=7,
            grid=(_T,),
            in_specs=[
                pl.BlockSpec((N, NFEAT), lambda t, *s: (0, 0)),
                pl.BlockSpec((ROWS, N), lambda t, *s: (s[0][t], 0)),
                pl.BlockSpec((RW2, CW2), lambda t, *s: (s[1][t], s[2][t])),
                pl.BlockSpec((NFEAT, NHID), lambda t, *s: (0, 0)),
                pl.BlockSpec((NHID, NCLASS), lambda t, *s: (0, 0)),
                pl.BlockSpec((1, NHID), lambda t, *s: (0, 0)),
                pl.BlockSpec((1, NCLASS), lambda t, *s: (0, 0)),
            ],
            out_specs=pl.BlockSpec(
                (RW2, NCLASS), lambda t, *s: (s[6][t], 0)),
            scratch_shapes=[
                pltpu.VMEM((N, NW), jnp.bfloat16),
                pltpu.VMEM((NG * RW2 + 32, NCLASS), jnp.bfloat16),
                pltpu.VMEM((NG * RW2 + 32, NCLASS), jnp.bfloat16),
                pltpu.VMEM((SROWS, EDGE), jnp.bfloat16),
            ],
        ),
        out_shape=jax.ShapeDtypeStruct((N, NCLASS), jnp.float32),
        compiler_params=pltpu.CompilerParams(
            dimension_semantics=("arbitrary",),
        ),
    )(jnp.asarray(_IA), jnp.asarray(_GB), jnp.asarray(_CB),
      jnp.asarray(_PH), jnp.asarray(_GG), jnp.asarray(_CC),
      jnp.asarray(_OG), x, adj, adj, W1, W2, b1r, b2r)

    return out
